# Initial kernel scaffold; baseline (speedup 1.0000x reference)
import jax
import jax.numpy as jnp
from jax.experimental import pallas as pl


def kernel(feat, graph_ids):
    raise NotImplementedError("write your pallas kernel here")


# speedup vs baseline: 3.0776x; 3.0776x over previous
import jax
import jax.numpy as jnp
from jax import lax
from jax.experimental import pallas as pl
from jax.experimental.pallas import tpu as pltpu
from jax.experimental.pallas import tpu_sc as plsc

N = 50000
D = 256
G = 128
NC = 2
NS = 16
NW = NC * NS
C = 1568
K = 112
NCH = C // K
GP = 136
NV = D // 16


def _seg_sum_body(feat_hbm, ids_hbm, part_hbm, pcnt_hbm,
                  buf, idv, accv, cntv):
    c = lax.axis_index("c")
    s = lax.axis_index("s")
    w = s * NC + c
    r0 = w * C
    r1 = jnp.minimum(r0 + C, N)

    zeros16 = jnp.zeros((16,), jnp.float32)
    ones16 = jnp.ones((16,), jnp.float32)
    iota16 = lax.iota(jnp.int32, 16)

    def zbody(g, carry):
        for k in range(NV):
            accv[g, pl.ds(k * 16, 16)] = zeros16
        cntv[g, :] = zeros16
        return carry

    lax.fori_loop(0, GP, zbody, 0)

    def chunk_body(ci, carry):
        p0 = r0 + ci * K
        p1 = jnp.minimum(p0 + K, r1)
        sdma = jnp.minimum(p0, N - K)
        pltpu.sync_copy(feat_hbm.at[pl.ds(sdma, K)], buf)
        pltpu.sync_copy(ids_hbm.at[pl.ds(sdma, K)], idv)

        for t in range(K // 16):
            rowv = jnp.full((16,), sdma + t * 16, jnp.int32) + iota16
            ok = (rowv >= jnp.full((16,), p0, jnp.int32)) & \
                 (rowv < jnp.full((16,), p1, jnp.int32))
            v = idv[pl.ds(t * 16, 16)]
            idv[pl.ds(t * 16, 16)] = jnp.where(ok, v,
                                               jnp.full((16,), G, jnp.int32))

        def rbody(r, carry):
            base = pl.multiple_of((r // 16) * 16, 16)
            idvec = idv[pl.ds(base, 16)]
            lanev = jnp.full((16,), r - base, jnp.int32)
            gidv = lax.gather(
                idvec, lanev[:, None],
                lax.GatherDimensionNumbers(offset_dims=(),
                                           collapsed_slice_dims=(0,),
                                           start_index_map=(0,)),
                slice_sizes=(1,),
                mode=lax.GatherScatterMode.PROMISE_IN_BOUNDS)
            for k in range(NV):
                plsc.addupdate_scatter(accv, [gidv, iota16 + (k * 16)],
                                       buf[r, pl.ds(k * 16, 16)])
            plsc.addupdate_scatter(cntv, [gidv, iota16], ones16)
            return carry

        lax.fori_loop(0, K, rbody, 0)
        return carry

    lax.fori_loop(0, NCH, chunk_body, 0)

    pltpu.sync_copy(accv, part_hbm.at[w])
    pltpu.sync_copy(cntv, pcnt_hbm.at[w])


def _seg_sum(feat, ids):
    mesh = plsc.VectorSubcoreMesh(core_axis_name="c", subcore_axis_name="s",
                                  num_cores=NC, num_subcores=NS)
    fn = pl.kernel(
        _seg_sum_body,
        out_type=(jax.ShapeDtypeStruct((NW, GP, D), jnp.float32),
                  jax.ShapeDtypeStruct((NW, GP, 16), jnp.float32)),
        mesh=mesh,
        compiler_params=pltpu.CompilerParams(needs_layout_passes=False),
        scratch_types=[
            pltpu.VMEM((K, D), jnp.float32),
            pltpu.VMEM((K,), jnp.int32),
            pltpu.VMEM((GP, D), jnp.float32),
            pltpu.VMEM((GP, 16), jnp.float32),
        ],
    )
    return fn(feat, ids)


def _finalize_body(part_ref, pc_ref, out_ref):
    def body(wi, a):
        return a + part_ref[wi, :G, :]

    acc = lax.fori_loop(0, NW, body, jnp.zeros((G, D), jnp.float32))

    def body2(wi, a):
        return a + pc_ref[wi, :G, :]

    cnt = lax.fori_loop(0, NW, body2, jnp.zeros((G, 16), jnp.float32))
    c1 = jnp.maximum(cnt[:, 0:1], 1.0)
    out_ref[...] = acc / c1


def _finalize(partials, pcnt):
    return pl.pallas_call(
        _finalize_body,
        out_shape=jax.ShapeDtypeStruct((G, D), jnp.float32),
    )(partials, pcnt)


@jax.jit
def kernel(feat, graph_ids):
    ids = graph_ids.astype(jnp.int32)
    partials, pcnt = _seg_sum(feat, ids)
    return _finalize(partials, pcnt)


# speedup vs baseline: 5.3701x; 1.7449x over previous
import jax
import jax.numpy as jnp
from jax import lax
from jax.experimental import pallas as pl
from jax.experimental.pallas import tpu as pltpu
from jax.experimental.pallas import tpu_sc as plsc

N = 50000
D = 256
G = 128
NC = 2
NS = 16
NW = NC * NS
C = 1568
K = 112
NCH = C // K
NV = D // 16

_GDN = lax.GatherDimensionNumbers(offset_dims=(), collapsed_slice_dims=(0,),
                                  start_index_map=(0,))


def _splat(vec, lane):
    idx = jnp.full((16,), lane, jnp.int32)
    return lax.gather(vec, idx[:, None], _GDN, slice_sizes=(1,),
                      mode=lax.GatherScatterMode.PROMISE_IN_BOUNDS)


def _seg_sum_body(feat_hbm, ids_hbm, part_hbm, pcnt_hbm, buf, idv, accv, cntv):
    c = lax.axis_index("c")
    s = lax.axis_index("s")
    w = s * NC + c
    r0 = w * C
    r1 = jnp.minimum(r0 + C, N)

    zeros16 = jnp.zeros((16,), jnp.float32)
    ones16 = jnp.ones((16,), jnp.float32)
    iota16 = lax.iota(jnp.int32, 16)
    cols = [iota16 + (k * 16) for k in range(NV)]

    def zbody(g, carry):
        for k in range(NV):
            accv[g, pl.ds(k * 16, 16)] = zeros16
        cntv[g, :] = zeros16
        return carry

    lax.fori_loop(0, G, zbody, 0)

    def chunk_body(ci, carry):
        prev, cntf, acc = carry
        p0 = r0 + ci * K
        p1 = jnp.minimum(p0 + K, r1)
        sdma = jnp.minimum(p0, N - K)
        pltpu.sync_copy(feat_hbm.at[pl.ds(sdma, K)], buf)
        pltpu.sync_copy(ids_hbm.at[pl.ds(sdma, K)], idv)

        def group_body(t, carry):
            prev, cntf, acc = carry
            idvec = idv[pl.ds(t * 16, 16)]
            for j in range(16):
                r = t * 16 + j
                gidv = _splat(idvec, j)
                rowi = sdma + r
                inr = ((rowi >= p0) & (rowi < p1)).astype(jnp.float32)
                inrv = jnp.full((16,), inr, jnp.float32)
                samev = jnp.where(gidv == prev, ones16, zeros16)
                acc = tuple(acc[k] * samev + buf[r, pl.ds(k * 16, 16)] * inrv
                            for k in range(NV))
                cntf = cntf * samev + inrv
                for k in range(NV):
                    plsc.store_scatter(accv, [gidv, cols[k]], acc[k])
                plsc.store_scatter(cntv, [gidv, iota16], cntf)
                prev = gidv
            return prev, cntf, acc

        return lax.fori_loop(0, K // 16, group_body, (prev, cntf, acc))

    init = (jnp.full((16,), -1, jnp.int32), zeros16,
            tuple(zeros16 for _ in range(NV)))
    lax.fori_loop(0, NCH, chunk_body, init)

    pltpu.sync_copy(accv, part_hbm.at[w])
    pltpu.sync_copy(cntv, pcnt_hbm.at[w])


def _seg_sum(feat, ids):
    mesh = plsc.VectorSubcoreMesh(core_axis_name="c", subcore_axis_name="s",
                                  num_cores=NC, num_subcores=NS)
    fn = pl.kernel(
        _seg_sum_body,
        out_type=(jax.ShapeDtypeStruct((NW, G, D), jnp.float32),
                  jax.ShapeDtypeStruct((NW, G, 16), jnp.float32)),
        mesh=mesh,
        compiler_params=pltpu.CompilerParams(needs_layout_passes=False),
        scratch_types=[
            pltpu.VMEM((K, D), jnp.float32),
            pltpu.VMEM((K,), jnp.int32),
            pltpu.VMEM((G, D), jnp.float32),
            pltpu.VMEM((G, 16), jnp.float32),
        ],
    )
    return fn(feat, ids)


def _finalize_body(part_ref, pc_ref, out_ref):
    def body(wi, a):
        return a + part_ref[wi]

    acc = lax.fori_loop(0, NW, body, jnp.zeros((G, D), jnp.float32))

    def body2(wi, a):
        return a + pc_ref[wi]

    cnt = lax.fori_loop(0, NW, body2, jnp.zeros((G, 16), jnp.float32))
    c1 = jnp.maximum(cnt[:, 0:1], 1.0)
    out_ref[...] = acc / c1


def _finalize(partials, pcnt):
    return pl.pallas_call(
        _finalize_body,
        out_shape=jax.ShapeDtypeStruct((G, D), jnp.float32),
    )(partials, pcnt)


@jax.jit
def kernel(feat, graph_ids):
    ids = graph_ids.astype(jnp.int32)
    partials, pcnt = _seg_sum(feat, ids)
    return _finalize(partials, pcnt)
